# baseline (speedup 1.0000x reference)
from functools import partial
from math import sqrt

import jax
import jax.numpy as jnp
import numpy as np
from jax import lax
from jax.experimental import pallas as pl
from jax.experimental.pallas import tpu as pltpu

_FACTOR = 5
_RB = 256
_G = 2

_CONST_CACHE = {}


def _tf2x32(k1, k2, x0, x1):
    rot1 = (13, 15, 26, 6)
    rot2 = (17, 29, 16, 24)

    def rnd(a, b, r):
        a = (a + b).astype(np.uint32)
        b = ((b << np.uint32(r)) | (b >> np.uint32(32 - r))).astype(np.uint32)
        return a, a ^ b

    ks = (np.uint32(k1), np.uint32(k2),
          np.uint32(np.uint32(k1) ^ np.uint32(k2) ^ np.uint32(0x1BD11BDA)))
    x0 = (x0 + ks[0]).astype(np.uint32)
    x1 = (x1 + ks[1]).astype(np.uint32)
    for i, rots in enumerate((rot1, rot2, rot1, rot2, rot1)):
        for r in rots:
            x0, x1 = rnd(x0, x1, r)
        x0 = (x0 + ks[(i + 1) % 3]).astype(np.uint32)
        x1 = (x1 + ks[(i + 2) % 3] + np.uint32(i + 1)).astype(np.uint32)
    return x0, x1


def _np_randint(seed, shape, span):
    k1 = np.uint32(np.uint64(seed) >> np.uint64(32))
    k2 = np.uint32(np.uint64(seed) & np.uint64(0xFFFFFFFF))
    o0, o1 = _tf2x32(k1, k2, np.zeros(2, np.uint32),
                     np.arange(2, dtype=np.uint32))
    n = int(np.prod(shape))
    i = np.arange(n, dtype=np.uint64)
    c1 = (i >> np.uint64(32)).astype(np.uint32)
    c2 = (i & np.uint64(0xFFFFFFFF)).astype(np.uint32)
    hb = _tf2x32(o0[0], o1[0], c1, c2)
    lb = _tf2x32(o0[1], o1[1], c1, c2)
    higher, lower = hb[0] ^ hb[1], lb[0] ^ lb[1]
    span = np.uint32(span)
    mult = np.uint32((int(2 ** 16 % int(span)) ** 2) % int(span))
    off = ((higher % span) * mult + lower % span) % span
    return off.reshape(shape).astype(np.int32)


def _sample_counts(L_Q, L_K, U_part):
    ck = (L_Q, L_K, U_part)
    if ck not in _CONST_CACHE:
        idx = _np_randint(42, (L_Q, U_part), L_K)
        cnt = np.zeros((L_Q, L_K), np.float32)
        np.add.at(cnt, (np.arange(L_Q)[:, None], idx), 1.0)
        mask = np.where(cnt > 0, 0.0, -1e30)
        _CONST_CACHE[ck] = (cnt.astype(np.dtype("bfloat16")),
                            mask.astype(np.dtype("bfloat16")))
    return _CONST_CACHE[ck]


def _body(q_ref, k_ref, v_ref, c_ref, mk_ref, o_ref, oh_ref, *, L, D, u, nb):
    idx_mat = (lax.broadcasted_iota(jnp.int32, (_RB, nb), 1) * _RB
               + lax.broadcasted_iota(jnp.int32, (_RB, nb), 0))
    iota_ul_i = lax.broadcasted_iota(jnp.int32, (u, L), 1)
    lt = (lax.broadcasted_iota(jnp.int32, (_RB, _RB), 0)
          >= lax.broadcasted_iota(jnp.int32, (_RB, _RB), 1)
          ).astype(jnp.float32)
    ones_u = jnp.ones((u, 1), jnp.float32)

    ps = []
    for g in range(_G):
        k = k_ref[g]
        ksum = jnp.dot(c_ref[...], k.astype(jnp.bfloat16),
                       preferred_element_type=jnp.float32)
        sumpart = jnp.sum(q_ref[g] * ksum, axis=1, keepdims=True)
        mcols = []
        for bi in range(nb):
            qb = q_ref[g, bi * _RB:(bi + 1) * _RB, :]
            s = lax.dot_general(qb, k, (((1,), (1,)), ((), ())),
                                preferred_element_type=jnp.float32)
            mkb = mk_ref[bi * _RB:(bi + 1) * _RB, :].astype(jnp.float32)
            mx = jnp.max(s + mkb, axis=1, keepdims=True)
            mcols.append(
                mx - sumpart[bi * _RB:(bi + 1) * _RB, :] * (1.0 / L))
        mt = jnp.concatenate(mcols, axis=1)
        bits = lax.bitcast_convert_type(mt, jnp.int32)
        key = jnp.where(bits >= 0, bits, bits ^ jnp.int32(0x7FFFFFFF))
        key = (key + jnp.int32(L // 2)) & jnp.int32(~(L - 1))
        ps.append(key | idx_mat)

    carries = [jnp.zeros((1, D), jnp.float32) for _ in range(_G)]
    for bi in range(nb):
        for g in range(_G):
            vb = v_ref[g, bi * _RB:(bi + 1) * _RB, :]
            o_ref[g, bi * _RB:(bi + 1) * _RB, :] = \
                jnp.dot(lt, vb, preferred_element_type=jnp.float32) \
                + carries[g]
            carries[g] = carries[g] + jnp.sum(vb, axis=0, keepdims=True)

    sels = [[] for _ in range(_G)]
    for j in range(u):
        for g in range(_G):
            pmax = jnp.max(ps[g], axis=(0, 1), keepdims=True)
            sels[g].append(pmax & jnp.int32(L - 1))
            ps[g] = jnp.where(ps[g] == pmax, jnp.int32(-(2 ** 31)), ps[g])
    gcols = [jnp.concatenate(s, axis=0) for s in sels]
    for g in range(_G):
        oh_ref[g] = (iota_ul_i == gcols[g]).astype(jnp.float32)

    upds = []
    for g in range(_G):
        oh = oh_ref[g]
        q_red = jnp.dot(oh, q_ref[g], preferred_element_type=jnp.float32)
        sc = lax.dot_general(q_red, k_ref[g], (((1,), (1,)), ((), ())),
                             preferred_element_type=jnp.float32)
        sc = sc * (1.0 / sqrt(D))
        sc = jnp.where(iota_ul_i > gcols[g], -1e30, sc)
        mrow = jnp.max(sc, axis=1, keepdims=True)
        e = jnp.exp(sc - mrow)
        attn = e * (1.0 / jnp.sum(e, axis=1, keepdims=True))
        upds.append(jnp.dot(attn, v_ref[g],
                            preferred_element_type=jnp.float32))

    for bi in range(nb):
        for g in range(_G):
            ohb = oh_ref[g, :, bi * _RB:(bi + 1) * _RB]
            scat = lax.dot_general(ohb, upds[g], (((0,), (0,)), ((), ())),
                                   preferred_element_type=jnp.float32)
            member = lax.dot_general(ohb, ones_u, (((0,), (0,)), ((), ())))
            o_ref[g, bi * _RB:(bi + 1) * _RB, :] = jnp.where(
                member > 0, scat, o_ref[g, bi * _RB:(bi + 1) * _RB, :])


def kernel(queries, keys, values):
    B, L_Q, H, D = queries.shape
    L_K = keys.shape[1]
    U_part = min(_FACTOR * int(np.ceil(np.log(L_K))), L_K)
    u = min(_FACTOR * int(np.ceil(np.log(L_Q))), L_Q)
    cnt, mask = _sample_counts(L_Q, L_K, U_part)
    cnt = jnp.asarray(cnt)
    mask = jnp.asarray(mask)

    bh = B * H
    nb = L_Q // _RB
    qt = queries.transpose(0, 2, 1, 3).reshape(bh, L_Q, D)
    kt = keys.transpose(0, 2, 1, 3).reshape(bh, L_K, D)
    vt = values.transpose(0, 2, 1, 3).reshape(bh, L_K, D)

    out = pl.pallas_call(
        partial(_body, L=L_K, D=D, u=u, nb=nb),
        grid=(bh // _G,),
        in_specs=[
            pl.BlockSpec((_G, L_Q, D), lambda i: (i, 0, 0)),
            pl.BlockSpec((_G, L_K, D), lambda i: (i, 0, 0)),
            pl.BlockSpec((_G, L_K, D), lambda i: (i, 0, 0)),
            pl.BlockSpec((L_Q, L_K), lambda i: (0, 0)),
            pl.BlockSpec((L_Q, L_K), lambda i: (0, 0)),
        ],
        out_specs=pl.BlockSpec((_G, L_Q, D), lambda i: (i, 0, 0)),
        out_shape=jax.ShapeDtypeStruct((bh, L_Q, D), jnp.float32),
        scratch_shapes=[
            pltpu.VMEM((_G, u, L_K), jnp.float32),
        ],
        compiler_params=pltpu.CompilerParams(
            dimension_semantics=("parallel",)),
    )(qt, kt, vt, cnt, mask)
    return out.reshape(B, H, L_Q, D)


# speedup vs baseline: 1.0505x; 1.0505x over previous
from functools import partial
from math import sqrt

import jax
import jax.numpy as jnp
import numpy as np
from jax import lax
from jax.experimental import pallas as pl
from jax.experimental.pallas import tpu as pltpu

_FACTOR = 5
_RB = 256
_G = 2

_CONST_CACHE = {}


def _tf2x32(k1, k2, x0, x1):
    rot1 = (13, 15, 26, 6)
    rot2 = (17, 29, 16, 24)

    def rnd(a, b, r):
        a = (a + b).astype(np.uint32)
        b = ((b << np.uint32(r)) | (b >> np.uint32(32 - r))).astype(np.uint32)
        return a, a ^ b

    ks = (np.uint32(k1), np.uint32(k2),
          np.uint32(np.uint32(k1) ^ np.uint32(k2) ^ np.uint32(0x1BD11BDA)))
    x0 = (x0 + ks[0]).astype(np.uint32)
    x1 = (x1 + ks[1]).astype(np.uint32)
    for i, rots in enumerate((rot1, rot2, rot1, rot2, rot1)):
        for r in rots:
            x0, x1 = rnd(x0, x1, r)
        x0 = (x0 + ks[(i + 1) % 3]).astype(np.uint32)
        x1 = (x1 + ks[(i + 2) % 3] + np.uint32(i + 1)).astype(np.uint32)
    return x0, x1


def _np_randint(seed, shape, span):
    k1 = np.uint32(np.uint64(seed) >> np.uint64(32))
    k2 = np.uint32(np.uint64(seed) & np.uint64(0xFFFFFFFF))
    o0, o1 = _tf2x32(k1, k2, np.zeros(2, np.uint32),
                     np.arange(2, dtype=np.uint32))
    n = int(np.prod(shape))
    i = np.arange(n, dtype=np.uint64)
    c1 = (i >> np.uint64(32)).astype(np.uint32)
    c2 = (i & np.uint64(0xFFFFFFFF)).astype(np.uint32)
    hb = _tf2x32(o0[0], o1[0], c1, c2)
    lb = _tf2x32(o0[1], o1[1], c1, c2)
    higher, lower = hb[0] ^ hb[1], lb[0] ^ lb[1]
    span = np.uint32(span)
    mult = np.uint32((int(2 ** 16 % int(span)) ** 2) % int(span))
    off = ((higher % span) * mult + lower % span) % span
    return off.reshape(shape).astype(np.int32)


def _sample_counts(L_Q, L_K, U_part):
    ck = (L_Q, L_K, U_part)
    if ck not in _CONST_CACHE:
        idx = _np_randint(42, (L_Q, U_part), L_K)
        cnt = np.zeros((L_Q, L_K), np.float32)
        np.add.at(cnt, (np.arange(L_Q)[:, None], idx), 1.0)
        mask = np.where(cnt > 0, 0.0, -1e30)
        _CONST_CACHE[ck] = (cnt.astype(np.dtype("bfloat16")),
                            mask.astype(np.dtype("bfloat16")))
    return _CONST_CACHE[ck]


def _body(q_ref, k_ref, v_ref, c_ref, mk_ref, o_ref, oh_ref, *, L, D, u, nb):
    idx_mat = (lax.broadcasted_iota(jnp.int32, (nb, _RB), 0) * _RB
               + lax.broadcasted_iota(jnp.int32, (nb, _RB), 1))
    iota_ul_i = lax.broadcasted_iota(jnp.int32, (u, L), 1)
    lt = (lax.broadcasted_iota(jnp.int32, (_RB, _RB), 0)
          >= lax.broadcasted_iota(jnp.int32, (_RB, _RB), 1)
          ).astype(jnp.float32)
    ones_u = jnp.ones((u, 1), jnp.float32)

    ps = []
    for g in range(_G):
        k = k_ref[g]
        ksum = jnp.dot(c_ref[...], k.astype(jnp.bfloat16),
                       preferred_element_type=jnp.float32)
        sumpart = jnp.sum(q_ref[g] * ksum, axis=1, keepdims=True)
        mcols = []
        for bi in range(nb):
            qb = q_ref[g, bi * _RB:(bi + 1) * _RB, :]
            s = lax.dot_general(qb, k, (((1,), (1,)), ((), ())),
                                preferred_element_type=jnp.float32)
            mkb = mk_ref[bi * _RB:(bi + 1) * _RB, :].astype(jnp.float32)
            mx = jnp.max(s + mkb, axis=1, keepdims=True)
            mcols.append(
                mx - sumpart[bi * _RB:(bi + 1) * _RB, :] * (1.0 / L))
        mt = jnp.transpose(jnp.concatenate(mcols, axis=1))
        scale = float(2 ** 31 // (64 * L))
        key = jnp.rint(jnp.clip(mt, -63.0, 63.0) * scale).astype(jnp.int32)
        ps.append(key * jnp.int32(L) + idx_mat)

    carries = [jnp.zeros((1, D), jnp.float32) for _ in range(_G)]
    for bi in range(nb):
        for g in range(_G):
            vb = v_ref[g, bi * _RB:(bi + 1) * _RB, :]
            o_ref[g, bi * _RB:(bi + 1) * _RB, :] = \
                jnp.dot(lt, vb, preferred_element_type=jnp.float32) \
                + carries[g]
            carries[g] = carries[g] + jnp.sum(vb, axis=0, keepdims=True)

    sels = [[] for _ in range(_G)]
    for j in range(u):
        for g in range(_G):
            pmax = jnp.max(ps[g], axis=(0, 1), keepdims=True)
            sels[g].append(pmax & jnp.int32(L - 1))
            ps[g] = jnp.where(ps[g] == pmax, jnp.int32(-(2 ** 31)), ps[g])
    gcols = [jnp.concatenate(s, axis=0) for s in sels]
    for g in range(_G):
        oh_ref[g] = (iota_ul_i == gcols[g]).astype(jnp.float32)

    upds = []
    for g in range(_G):
        oh = oh_ref[g]
        q_red = jnp.dot(oh, q_ref[g], preferred_element_type=jnp.float32)
        sc = lax.dot_general(q_red, k_ref[g], (((1,), (1,)), ((), ())),
                             preferred_element_type=jnp.float32)
        sc = sc * (1.0 / sqrt(D))
        sc = jnp.where(iota_ul_i > gcols[g], -1e30, sc)
        mrow = jnp.max(sc, axis=1, keepdims=True)
        e = jnp.exp(sc - mrow)
        attn = e * (1.0 / jnp.sum(e, axis=1, keepdims=True))
        upds.append(jnp.dot(attn, v_ref[g],
                            preferred_element_type=jnp.float32))

    for bi in range(nb):
        for g in range(_G):
            ohb = oh_ref[g, :, bi * _RB:(bi + 1) * _RB]
            scat = lax.dot_general(ohb, upds[g], (((0,), (0,)), ((), ())),
                                   preferred_element_type=jnp.float32)
            member = lax.dot_general(ohb, ones_u, (((0,), (0,)), ((), ())))
            o_ref[g, bi * _RB:(bi + 1) * _RB, :] = jnp.where(
                member > 0, scat, o_ref[g, bi * _RB:(bi + 1) * _RB, :])


def kernel(queries, keys, values):
    B, L_Q, H, D = queries.shape
    L_K = keys.shape[1]
    U_part = min(_FACTOR * int(np.ceil(np.log(L_K))), L_K)
    u = min(_FACTOR * int(np.ceil(np.log(L_Q))), L_Q)
    cnt, mask = _sample_counts(L_Q, L_K, U_part)
    cnt = jnp.asarray(cnt)
    mask = jnp.asarray(mask)

    bh = B * H
    nb = L_Q // _RB
    qt = queries.transpose(0, 2, 1, 3).reshape(bh, L_Q, D)
    kt = keys.transpose(0, 2, 1, 3).reshape(bh, L_K, D)
    vt = values.transpose(0, 2, 1, 3).reshape(bh, L_K, D)

    out = pl.pallas_call(
        partial(_body, L=L_K, D=D, u=u, nb=nb),
        grid=(bh // _G,),
        in_specs=[
            pl.BlockSpec((_G, L_Q, D), lambda i: (i, 0, 0)),
            pl.BlockSpec((_G, L_K, D), lambda i: (i, 0, 0)),
            pl.BlockSpec((_G, L_K, D), lambda i: (i, 0, 0)),
            pl.BlockSpec((L_Q, L_K), lambda i: (0, 0)),
            pl.BlockSpec((L_Q, L_K), lambda i: (0, 0)),
        ],
        out_specs=pl.BlockSpec((_G, L_Q, D), lambda i: (i, 0, 0)),
        out_shape=jax.ShapeDtypeStruct((bh, L_Q, D), jnp.float32),
        scratch_shapes=[
            pltpu.VMEM((_G, u, L_K), jnp.float32),
        ],
        compiler_params=pltpu.CompilerParams(
            dimension_semantics=("parallel",)),
    )(qt, kt, vt, cnt, mask)
    return out.reshape(B, H, L_Q, D)
